# baseline (speedup 1.0000x reference)
import functools

import jax
import jax.numpy as jnp
from jax import lax
from jax.experimental import pallas as pl
from jax.experimental.pallas import tpu as pltpu
from jax.experimental.pallas import tpu_sc as plsc


def _sc_gather(xs, weight):
    B = xs.shape[0]
    V, D = weight.shape
    info = plsc.get_sparse_core_info()
    nc, ns = info.num_cores, info.num_subcores
    nw = nc * ns
    b_per_w = B // nw

    mesh = plsc.VectorSubcoreMesh(core_axis_name="c", subcore_axis_name="s")

    @functools.partial(
        pl.kernel,
        mesh=mesh,
        out_type=jax.ShapeDtypeStruct((B, D), jnp.float32),
        scratch_types=[
            pltpu.VMEM((b_per_w,), jnp.int32),
            pltpu.VMEM((b_per_w, D), jnp.float32),
            pltpu.SemaphoreType.DMA,
        ],
    )
    def gather_kernel(xs_hbm, w_hbm, out_hbm, idx_v, rows_v, sem):
        wid = lax.axis_index("s") * nc + lax.axis_index("c")
        base = wid * b_per_w
        pltpu.sync_copy(xs_hbm.at[pl.ds(base, b_per_w)], idx_v)
        pltpu.async_copy(w_hbm.at[idx_v], rows_v, sem).wait()
        pltpu.sync_copy(rows_v, out_hbm.at[pl.ds(base, b_per_w)])

    return gather_kernel(xs, weight)


def _tc_project_t(emb, weight, vocab_block=4000, ring=3):
    B, D = emb.shape
    V = weight.shape[0]
    VB, S = vocab_block, ring
    assert V % VB == 0 and VB % 8 == 0
    nsteps = V // VB

    def body(emb_ref, w_ref, out_hbm, bufs, sems):
        g = pl.program_id(0)
        b = lax.rem(g, S)

        @pl.when(g >= S)
        def _():
            pltpu.make_async_copy(bufs.at[b], out_hbm.at[pl.ds((g - S) * VB, VB)],
                                  sems.at[b]).wait()

        a = emb_ref[...].astype(jnp.bfloat16)
        w = w_ref[...].astype(jnp.bfloat16)
        bufs[b] = lax.dot_general(
            w, a, (((1,), (1,)), ((), ())),
            preferred_element_type=jnp.float32,
        )

        for k in range(S):
            @pl.when(b == k)
            def _(k=k):
                pltpu.make_async_copy(bufs.at[k],
                                      out_hbm.at[pl.ds(g * VB, VB)],
                                      sems.at[k]).start(priority=k % 2)

        @pl.when(g == nsteps - 1)
        def _():
            for k in range(S):
                gk = nsteps - 1 - k
                bk = gk % S
                pltpu.make_async_copy(bufs.at[bk],
                                      out_hbm.at[pl.ds(gk * VB, VB)],
                                      sems.at[bk]).wait()

    return pl.pallas_call(
        body,
        grid=(nsteps,),
        in_specs=[
            pl.BlockSpec((B, D), lambda i: (0, 0)),
            pl.BlockSpec((VB, D), lambda i: (i, 0)),
        ],
        out_specs=pl.BlockSpec(memory_space=pl.ANY),
        out_shape=jax.ShapeDtypeStruct((V, B), jnp.float32),
        scratch_shapes=[
            pltpu.VMEM((S, VB, B), jnp.float32),
            pltpu.SemaphoreType.DMA((S,)),
        ],
    )(emb, weight)


def kernel(xs, weight):
    emb = _sc_gather(xs.astype(jnp.int32), weight)
    out_t = _tc_project_t(emb, weight)
    return out_t.T


# speedup vs baseline: 1.1116x; 1.1116x over previous
import functools

import jax
import jax.numpy as jnp
from jax import lax
from jax.experimental import pallas as pl
from jax.experimental.pallas import tpu as pltpu
from jax.experimental.pallas import tpu_sc as plsc


def _sc_gather(xs, weight):
    B = xs.shape[0]
    V, D = weight.shape
    info = plsc.get_sparse_core_info()
    nc, ns = info.num_cores, info.num_subcores
    nw = nc * ns
    b_per_w = B // nw

    mesh = plsc.VectorSubcoreMesh(core_axis_name="c", subcore_axis_name="s")

    @functools.partial(
        pl.kernel,
        mesh=mesh,
        out_type=jax.ShapeDtypeStruct((B, D), jnp.float32),
        scratch_types=[
            pltpu.VMEM((b_per_w,), jnp.int32),
            pltpu.VMEM((b_per_w, D), jnp.float32),
            pltpu.SemaphoreType.DMA,
        ],
    )
    def gather_kernel(xs_hbm, w_hbm, out_hbm, idx_v, rows_v, sem):
        wid = lax.axis_index("s") * nc + lax.axis_index("c")
        base = wid * b_per_w
        pltpu.sync_copy(xs_hbm.at[pl.ds(base, b_per_w)], idx_v)
        pltpu.async_copy(w_hbm.at[idx_v], rows_v, sem).wait()
        pltpu.sync_copy(rows_v, out_hbm.at[pl.ds(base, b_per_w)])

    return gather_kernel(xs, weight)


def _tc_project_t(emb, weight, vocab_block=2000, ring=4):
    B, D = emb.shape
    V = weight.shape[0]
    VB, S = vocab_block, ring
    assert V % VB == 0 and VB % 8 == 0
    nsteps = V // VB

    def body(emb_ref, w_ref, out_hbm, bufs, sems):
        g = pl.program_id(0)
        b = lax.rem(g, S)

        @pl.when(g >= S)
        def _():
            pltpu.make_async_copy(bufs.at[b], out_hbm.at[pl.ds((g - S) * VB, VB)],
                                  sems.at[b]).wait()

        a = emb_ref[...].astype(jnp.bfloat16)
        w = w_ref[...].astype(jnp.bfloat16)
        bufs[b] = lax.dot_general(
            w, a, (((1,), (1,)), ((), ())),
            preferred_element_type=jnp.float32,
        )

        for k in range(S):
            @pl.when(b == k)
            def _(k=k):
                pltpu.make_async_copy(bufs.at[k],
                                      out_hbm.at[pl.ds(g * VB, VB)],
                                      sems.at[k]).start(priority=1)

        @pl.when(g == nsteps - 1)
        def _():
            for k in range(S):
                gk = nsteps - 1 - k
                bk = gk % S
                pltpu.make_async_copy(bufs.at[bk],
                                      out_hbm.at[pl.ds(gk * VB, VB)],
                                      sems.at[bk]).wait()

    return pl.pallas_call(
        body,
        grid=(nsteps,),
        in_specs=[
            pl.BlockSpec((B, D), lambda i: (0, 0)),
            pl.BlockSpec((VB, D), lambda i: (i, 0)),
        ],
        out_specs=pl.BlockSpec(memory_space=pl.ANY),
        out_shape=jax.ShapeDtypeStruct((V, B), jnp.float32),
        scratch_shapes=[
            pltpu.VMEM((S, VB, B), jnp.float32),
            pltpu.SemaphoreType.DMA((S,)),
        ],
    )(emb, weight)


def kernel(xs, weight):
    emb = _sc_gather(xs.astype(jnp.int32), weight)
    out_t = _tc_project_t(emb, weight)
    return out_t.T


# speedup vs baseline: 1.1160x; 1.0040x over previous
import functools

import jax
import jax.numpy as jnp
from jax import lax
from jax.experimental import pallas as pl
from jax.experimental.pallas import tpu as pltpu
from jax.experimental.pallas import tpu_sc as plsc


def _sc_gather(xs, weight):
    B = xs.shape[0]
    V, D = weight.shape
    info = plsc.get_sparse_core_info()
    nc, ns = info.num_cores, info.num_subcores
    nw = nc * ns
    b_per_w = B // nw

    mesh = plsc.VectorSubcoreMesh(core_axis_name="c", subcore_axis_name="s")

    half = b_per_w // 2

    @functools.partial(
        pl.kernel,
        mesh=mesh,
        out_type=jax.ShapeDtypeStruct((B, D), jnp.float32),
        scratch_types=[
            pltpu.VMEM((b_per_w,), jnp.int32),
            pltpu.VMEM((b_per_w, D), jnp.float32),
            pltpu.SemaphoreType.DMA((4,)),
        ],
    )
    def gather_kernel(xs_hbm, w_hbm, out_hbm, idx_v, rows_v, sems):
        wid = lax.axis_index("s") * nc + lax.axis_index("c")
        base = wid * b_per_w
        pltpu.sync_copy(xs_hbm.at[pl.ds(base, b_per_w)], idx_v)
        g0 = pltpu.make_async_copy(w_hbm.at[idx_v.at[pl.ds(0, half)]],
                                   rows_v.at[pl.ds(0, half)], sems.at[0])
        g1 = pltpu.make_async_copy(w_hbm.at[idx_v.at[pl.ds(half, half)]],
                                   rows_v.at[pl.ds(half, half)], sems.at[1])
        g0.start()
        g1.start()
        g0.wait()
        o0 = pltpu.make_async_copy(rows_v.at[pl.ds(0, half)],
                                   out_hbm.at[pl.ds(base, half)], sems.at[2])
        o0.start()
        g1.wait()
        o1 = pltpu.make_async_copy(rows_v.at[pl.ds(half, half)],
                                   out_hbm.at[pl.ds(base + half, half)],
                                   sems.at[3])
        o1.start()
        o0.wait()
        o1.wait()

    return gather_kernel(xs, weight)


def _tc_project_t(emb, weight, vocab_block=2000, ring=4):
    B, D = emb.shape
    V = weight.shape[0]
    VB, S = vocab_block, ring
    assert V % VB == 0 and VB % 8 == 0
    nsteps = V // VB

    def body(emb_ref, w_ref, out_hbm, bufs, sems):
        g = pl.program_id(0)
        b = lax.rem(g, S)

        @pl.when(g >= S)
        def _():
            pltpu.make_async_copy(bufs.at[b], out_hbm.at[pl.ds((g - S) * VB, VB)],
                                  sems.at[b]).wait()

        a = emb_ref[...].astype(jnp.bfloat16)

        QC = 5
        qvb = VB // QC

        @pl.when(g == 0)
        def _():
            for q in range(QC):
                wq = w_ref[pl.ds(q * qvb, qvb), :].astype(jnp.bfloat16)
                bufs[0, pl.ds(q * qvb, qvb)] = lax.dot_general(
                    wq, a, (((1,), (1,)), ((), ())),
                    preferred_element_type=jnp.float32,
                )
                pltpu.make_async_copy(bufs.at[0, pl.ds(q * qvb, qvb)],
                                      out_hbm.at[pl.ds(q * qvb, qvb)],
                                      sems.at[0]).start(priority=1)

        @pl.when(g > 0)
        def _():
            w = w_ref[...].astype(jnp.bfloat16)
            bufs[b] = lax.dot_general(
                w, a, (((1,), (1,)), ((), ())),
                preferred_element_type=jnp.float32,
            )

        for k in range(S):
            @pl.when(jnp.logical_and(b == k, g > 0))
            def _(k=k):
                pltpu.make_async_copy(bufs.at[k],
                                      out_hbm.at[pl.ds(g * VB, VB)],
                                      sems.at[k]).start(priority=1)

        @pl.when(g == nsteps - 1)
        def _():
            for k in range(S):
                gk = nsteps - 1 - k
                bk = gk % S
                pltpu.make_async_copy(bufs.at[bk],
                                      out_hbm.at[pl.ds(gk * VB, VB)],
                                      sems.at[bk]).wait()

    return pl.pallas_call(
        body,
        grid=(nsteps,),
        in_specs=[
            pl.BlockSpec((B, D), lambda i: (0, 0)),
            pl.BlockSpec((VB, D), lambda i: (i, 0)),
        ],
        out_specs=pl.BlockSpec(memory_space=pl.ANY),
        out_shape=jax.ShapeDtypeStruct((V, B), jnp.float32),
        scratch_shapes=[
            pltpu.VMEM((S, VB, B), jnp.float32),
            pltpu.SemaphoreType.DMA((S,)),
        ],
    )(emb, weight)


def kernel(xs, weight):
    emb = _sc_gather(xs.astype(jnp.int32), weight)
    out_t = _tc_project_t(emb, weight)
    return out_t.T


# speedup vs baseline: 1.1180x; 1.0018x over previous
import functools

import jax
import jax.numpy as jnp
from jax import lax
from jax.experimental import pallas as pl
from jax.experimental.pallas import tpu as pltpu
from jax.experimental.pallas import tpu_sc as plsc


def _sc_gather(xs, weight):
    B = xs.shape[0]
    V, D = weight.shape
    info = plsc.get_sparse_core_info()
    nc, ns = info.num_cores, info.num_subcores
    nw = nc * ns
    b_per_w = B // nw

    mesh = plsc.VectorSubcoreMesh(core_axis_name="c", subcore_axis_name="s")

    half = b_per_w // 2

    @functools.partial(
        pl.kernel,
        mesh=mesh,
        out_type=jax.ShapeDtypeStruct((B, D), jnp.float32),
        scratch_types=[
            pltpu.VMEM((b_per_w,), jnp.int32),
            pltpu.VMEM((b_per_w, D), jnp.float32),
            pltpu.SemaphoreType.DMA((4,)),
        ],
    )
    def gather_kernel(xs_hbm, w_hbm, out_hbm, idx_v, rows_v, sems):
        wid = lax.axis_index("s") * nc + lax.axis_index("c")
        base = wid * b_per_w
        pltpu.sync_copy(xs_hbm.at[pl.ds(base, b_per_w)], idx_v)
        g0 = pltpu.make_async_copy(w_hbm.at[idx_v.at[pl.ds(0, half)]],
                                   rows_v.at[pl.ds(0, half)], sems.at[0])
        g1 = pltpu.make_async_copy(w_hbm.at[idx_v.at[pl.ds(half, half)]],
                                   rows_v.at[pl.ds(half, half)], sems.at[1])
        g0.start()
        g1.start()
        g0.wait()
        o0 = pltpu.make_async_copy(rows_v.at[pl.ds(0, half)],
                                   out_hbm.at[pl.ds(base, half)], sems.at[2])
        o0.start()
        g1.wait()
        o1 = pltpu.make_async_copy(rows_v.at[pl.ds(half, half)],
                                   out_hbm.at[pl.ds(base + half, half)],
                                   sems.at[3])
        o1.start()
        o0.wait()
        o1.wait()

    return gather_kernel(xs, weight)


def _tc_project_t(emb, weight, vocab_block=2000, ring=4):
    B, D = emb.shape
    V = weight.shape[0]
    VB, S = vocab_block, ring
    assert V % VB == 0 and VB % 8 == 0
    nsteps = V // VB

    def body(emb_ref, w_ref, out_hbm, bufs, sems):
        g = pl.program_id(0)
        b = lax.rem(g, S)

        @pl.when(g >= S)
        def _():
            pltpu.make_async_copy(bufs.at[b], out_hbm.at[pl.ds((g - S) * VB, VB)],
                                  sems.at[b]).wait()

        a = emb_ref[...].astype(jnp.bfloat16)

        QC = 10
        qvb = VB // QC

        @pl.when(g == 0)
        def _():
            for q in range(QC):
                wq = w_ref[pl.ds(q * qvb, qvb), :].astype(jnp.bfloat16)
                bufs[0, pl.ds(q * qvb, qvb)] = lax.dot_general(
                    wq, a, (((1,), (1,)), ((), ())),
                    preferred_element_type=jnp.float32,
                )
                pltpu.make_async_copy(bufs.at[0, pl.ds(q * qvb, qvb)],
                                      out_hbm.at[pl.ds(q * qvb, qvb)],
                                      sems.at[0]).start(priority=1)

        @pl.when(g > 0)
        def _():
            w = w_ref[...].astype(jnp.bfloat16)
            bufs[b] = lax.dot_general(
                w, a, (((1,), (1,)), ((), ())),
                preferred_element_type=jnp.float32,
            )

        for k in range(S):
            @pl.when(jnp.logical_and(b == k, g > 0))
            def _(k=k):
                pltpu.make_async_copy(bufs.at[k],
                                      out_hbm.at[pl.ds(g * VB, VB)],
                                      sems.at[k]).start(priority=1)

        @pl.when(g == nsteps - 1)
        def _():
            for k in range(S):
                gk = nsteps - 1 - k
                bk = gk % S
                pltpu.make_async_copy(bufs.at[bk],
                                      out_hbm.at[pl.ds(gk * VB, VB)],
                                      sems.at[bk]).wait()

    return pl.pallas_call(
        body,
        grid=(nsteps,),
        in_specs=[
            pl.BlockSpec((B, D), lambda i: (0, 0)),
            pl.BlockSpec((VB, D), lambda i: (i, 0)),
        ],
        out_specs=pl.BlockSpec(memory_space=pl.ANY),
        out_shape=jax.ShapeDtypeStruct((V, B), jnp.float32),
        scratch_shapes=[
            pltpu.VMEM((S, VB, B), jnp.float32),
            pltpu.SemaphoreType.DMA((S,)),
        ],
    )(emb, weight)


def kernel(xs, weight):
    emb = _sc_gather(xs.astype(jnp.int32), weight)
    out_t = _tc_project_t(emb, weight)
    return out_t.T


# speedup vs baseline: 1.1182x; 1.0002x over previous
import functools

import jax
import jax.numpy as jnp
from jax import lax
from jax.experimental import pallas as pl
from jax.experimental.pallas import tpu as pltpu
from jax.experimental.pallas import tpu_sc as plsc


def _sc_gather(xs, weight):
    B = xs.shape[0]
    V, D = weight.shape
    info = plsc.get_sparse_core_info()
    nc, ns = info.num_cores, info.num_subcores
    nw = nc * ns
    b_per_w = B // nw

    mesh = plsc.VectorSubcoreMesh(core_axis_name="c", subcore_axis_name="s")

    half = b_per_w // 2

    @functools.partial(
        pl.kernel,
        mesh=mesh,
        out_type=jax.ShapeDtypeStruct((B, D), jnp.float32),
        scratch_types=[
            pltpu.VMEM((b_per_w,), jnp.int32),
            pltpu.VMEM((b_per_w, D), jnp.float32),
            pltpu.SemaphoreType.DMA((4,)),
        ],
    )
    def gather_kernel(xs_hbm, w_hbm, out_hbm, idx_v, rows_v, sems):
        wid = lax.axis_index("s") * nc + lax.axis_index("c")
        base = wid * b_per_w
        pltpu.sync_copy(xs_hbm.at[pl.ds(base, b_per_w)], idx_v)
        g0 = pltpu.make_async_copy(w_hbm.at[idx_v.at[pl.ds(0, half)]],
                                   rows_v.at[pl.ds(0, half)], sems.at[0])
        g1 = pltpu.make_async_copy(w_hbm.at[idx_v.at[pl.ds(half, half)]],
                                   rows_v.at[pl.ds(half, half)], sems.at[1])
        g0.start()
        g1.start()
        g0.wait()
        o0 = pltpu.make_async_copy(rows_v.at[pl.ds(0, half)],
                                   out_hbm.at[pl.ds(base, half)], sems.at[2])
        o0.start()
        g1.wait()
        o1 = pltpu.make_async_copy(rows_v.at[pl.ds(half, half)],
                                   out_hbm.at[pl.ds(base + half, half)],
                                   sems.at[3])
        o1.start()
        o0.wait()
        o1.wait()

    return gather_kernel(xs, weight)


def _tc_project_t(emb, weight, vocab_block=2000, ring=4):
    B, D = emb.shape
    V = weight.shape[0]
    VB, S = vocab_block, ring
    assert V % VB == 0 and VB % 8 == 0
    nsteps = V // VB

    def body(emb_ref, w_ref, out_hbm, bufs, sems):
        g = pl.program_id(0)
        b = lax.rem(g, S)

        @pl.when(g >= S)
        def _():
            pltpu.make_async_copy(bufs.at[b], out_hbm.at[pl.ds((g - S) * VB, VB)],
                                  sems.at[b]).wait()

        a = emb_ref[...].astype(jnp.bfloat16)

        QC = 10
        qvb = VB // QC
        RAMP = 2

        @pl.when(g < RAMP)
        def _():
            for q in range(QC):
                wq = w_ref[pl.ds(q * qvb, qvb), :].astype(jnp.bfloat16)
                bufs[b, pl.ds(q * qvb, qvb)] = lax.dot_general(
                    wq, a, (((1,), (1,)), ((), ())),
                    preferred_element_type=jnp.float32,
                )
                pltpu.make_async_copy(bufs.at[b, pl.ds(q * qvb, qvb)],
                                      out_hbm.at[pl.ds(g * VB + q * qvb, qvb)],
                                      sems.at[b]).start(priority=1)

        @pl.when(g >= RAMP)
        def _():
            w = w_ref[...].astype(jnp.bfloat16)
            bufs[b] = lax.dot_general(
                w, a, (((1,), (1,)), ((), ())),
                preferred_element_type=jnp.float32,
            )

        for k in range(S):
            @pl.when(jnp.logical_and(b == k, g >= RAMP))
            def _(k=k):
                pltpu.make_async_copy(bufs.at[k],
                                      out_hbm.at[pl.ds(g * VB, VB)],
                                      sems.at[k]).start(priority=1)

        @pl.when(g == nsteps - 1)
        def _():
            for k in range(S):
                gk = nsteps - 1 - k
                bk = gk % S
                pltpu.make_async_copy(bufs.at[bk],
                                      out_hbm.at[pl.ds(gk * VB, VB)],
                                      sems.at[bk]).wait()

    return pl.pallas_call(
        body,
        grid=(nsteps,),
        in_specs=[
            pl.BlockSpec((B, D), lambda i: (0, 0)),
            pl.BlockSpec((VB, D), lambda i: (i, 0)),
        ],
        out_specs=pl.BlockSpec(memory_space=pl.ANY),
        out_shape=jax.ShapeDtypeStruct((V, B), jnp.float32),
        scratch_shapes=[
            pltpu.VMEM((S, VB, B), jnp.float32),
            pltpu.SemaphoreType.DMA((S,)),
        ],
    )(emb, weight)


def kernel(xs, weight):
    emb = _sc_gather(xs.astype(jnp.int32), weight)
    out_t = _tc_project_t(emb, weight)
    return out_t.T
